# Initial kernel scaffold; baseline (speedup 1.0000x reference)
import jax
import jax.numpy as jnp
from jax.experimental import pallas as pl


def kernel(x, edge_index, W0, b0, W1, b1, W2, b2):
    raise NotImplementedError("write your pallas kernel here")


# speedup vs baseline: 3.3296x; 3.3296x over previous
import functools

import jax
import jax.numpy as jnp
from jax import lax
from jax.experimental import pallas as pl
from jax.experimental.pallas import tpu as pltpu
from jax.experimental.pallas import tpu_sc as plsc

N = 10000
E = 160000
D = 256
DH = 128

_INFO = plsc.get_sparse_core_info()
NC = _INFO.num_cores
NS = _INFO.num_subcores
EPT = E // NS
RPT = 624
REM = N - NS * RPT
CHUNK = 80
NCH = EPT // CHUNK


def _sc_agg_body(src2_hbm, dst_hbm, h2_hbm, zeros_hbm, out_hbm,
                 idx_s, idx_d, rows, acc, sem):
    c = lax.axis_index("c")
    s = lax.axis_index("s")

    pltpu.sync_copy(zeros_hbm.at[pl.ds(0, RPT)], acc.at[pl.ds(s * RPT, RPT)])

    @pl.when(s == NS - 1)
    def _():
        pltpu.sync_copy(zeros_hbm.at[pl.ds(0, REM)],
                        acc.at[pl.ds(NS * RPT, REM)])

    plsc.subcore_barrier()

    def step(i, carry):
        base = s * EPT + i * CHUNK
        pltpu.sync_copy(src2_hbm.at[pl.ds(c * E + base, CHUNK)], idx_s)
        pltpu.sync_copy(dst_hbm.at[pl.ds(base, CHUNK)], idx_d)
        pltpu.async_copy(h2_hbm.at[idx_s], rows, sem).wait()
        pltpu.sync_copy(rows, acc.at[idx_d], add=True)
        return carry

    lax.fori_loop(0, NCH, step, 0)
    plsc.subcore_barrier()
    pltpu.sync_copy(acc.at[pl.ds(s * RPT, RPT)],
                    out_hbm.at[c, pl.ds(s * RPT, RPT)])

    @pl.when(s == NS - 1)
    def _():
        pltpu.sync_copy(acc.at[pl.ds(NS * RPT, REM)],
                        out_hbm.at[c, pl.ds(NS * RPT, REM)])


_sc_agg = pl.kernel(
    _sc_agg_body,
    out_type=jax.ShapeDtypeStruct((NC, N, DH), jnp.float32),
    mesh=plsc.VectorSubcoreMesh(core_axis_name="c", subcore_axis_name="s"),
    scratch_types=[
        pltpu.VMEM((CHUNK,), jnp.int32),
        pltpu.VMEM((CHUNK,), jnp.int32),
        pltpu.VMEM((CHUNK, DH), jnp.float32),
        pltpu.VMEM_SHARED((N, DH), jnp.float32),
        pltpu.SemaphoreType.DMA,
    ],
)


BM = 1000


def _mm_body(h_ref, a0_ref, a1_ref, w_ref, b_ref, o_ref):
    agg = jnp.concatenate([a0_ref[...], a1_ref[...]], axis=1)
    s = h_ref[...] + agg
    o_ref[...] = (
        jnp.dot(s, w_ref[...], preferred_element_type=jnp.float32) + b_ref[...]
    )


@functools.partial(jax.jit, static_argnames=())
def _tc_mm(h, a0, a1, w, b2d):
    return pl.pallas_call(
        _mm_body,
        grid=(N // BM,),
        in_specs=[
            pl.BlockSpec((BM, D), lambda i: (i, 0)),
            pl.BlockSpec((BM, DH), lambda i: (i, 0)),
            pl.BlockSpec((BM, DH), lambda i: (i, 0)),
            pl.BlockSpec((D, D), lambda i: (0, 0)),
            pl.BlockSpec((1, D), lambda i: (0, 0)),
        ],
        out_specs=pl.BlockSpec((BM, D), lambda i: (i, 0)),
        out_shape=jax.ShapeDtypeStruct((N, D), jnp.float32),
    )(h, a0, a1, w, b2d)


def kernel(x, edge_index, W0, b0, W1, b1, W2, b2):
    src = edge_index[0].astype(jnp.int32)
    dst = edge_index[1].astype(jnp.int32)
    src2 = jnp.concatenate([src * 2, src * 2 + 1])
    zeros = jnp.zeros((RPT, DH), jnp.float32)

    h = x
    for W, b in ((W0, b0), (W1, b1), (W2, b2)):
        h2 = h.reshape(2 * N, DH)
        agg = _sc_agg(src2, dst, h2, zeros)
        h = _tc_mm(h, agg[0], agg[1], W, b.reshape(1, D))
    return h


# speedup vs baseline: 7.3191x; 2.1982x over previous
import functools

import jax
import jax.numpy as jnp
from jax import lax
from jax.experimental import pallas as pl
from jax.experimental.pallas import tpu as pltpu
from jax.experimental.pallas import tpu_sc as plsc

N = 10000
E = 160000
D = 256
DH = 128

_INFO = plsc.get_sparse_core_info()
NC = _INFO.num_cores
NS = _INFO.num_subcores
EPT = E // NS
RPT = 624
REM = N - NS * RPT
CHUNK = 80
NCH = EPT // CHUNK


def _sc_agg_body(src2_hbm, dst_hbm, h2_hbm, zeros_hbm, out_hbm,
                 idx_s, db0, db1, rows, acc, sg0, sg1, sd0, sd1):
    c = lax.axis_index("c")
    s = lax.axis_index("s")
    sg = (sg0, sg1)
    sd = (sd0, sd1)
    db = (db0, db1)

    pltpu.sync_copy(zeros_hbm.at[pl.ds(0, RPT)], acc.at[pl.ds(s * RPT, RPT)])

    @pl.when(s == NS - 1)
    def _():
        pltpu.sync_copy(zeros_hbm.at[pl.ds(0, REM)],
                        acc.at[pl.ds(NS * RPT, REM)])

    pltpu.sync_copy(src2_hbm.at[pl.ds(c * E + s * EPT, EPT)], idx_s)
    plsc.subcore_barrier()

    def gather_start(i, b):
        pltpu.async_copy(h2_hbm.at[idx_s.at[pl.ds(i * CHUNK, CHUNK)]],
                         rows.at[b], sg[b])

    def gather_wait(b):
        pltpu.make_async_copy(h2_hbm.at[idx_s.at[pl.ds(0, CHUNK)]],
                              rows.at[b], sg[b]).wait()

    def dst_start(i, b):
        pltpu.async_copy(dst_hbm.at[pl.ds(s * EPT + i * CHUNK, CHUNK)],
                         db[b], sd[b])

    def dst_wait(b):
        pltpu.make_async_copy(dst_hbm.at[pl.ds(0, CHUNK)], db[b],
                              sd[b]).wait()

    def scatter(b):
        pltpu.sync_copy(rows.at[b], acc.at[db[b]], add=True)

    pltpu.sync_copy(dst_hbm.at[pl.ds(s * EPT, CHUNK)], db0)
    gather_start(0, 0)

    def step(k, carry):
        i0 = 2 * k
        dst_start(i0 + 1, 1)
        gather_start(i0 + 1, 1)
        gather_wait(0)
        scatter(0)
        dst_start(i0 + 2, 0)
        gather_start(i0 + 2, 0)
        dst_wait(1)
        gather_wait(1)
        scatter(1)
        dst_wait(0)
        return carry

    lax.fori_loop(0, (NCH - 1) // 2, step, 0)
    gather_wait(0)
    scatter(0)

    plsc.subcore_barrier()
    pltpu.sync_copy(acc.at[pl.ds(s * RPT, RPT)],
                    out_hbm.at[c, pl.ds(s * RPT, RPT)])

    @pl.when(s == NS - 1)
    def _():
        pltpu.sync_copy(acc.at[pl.ds(NS * RPT, REM)],
                        out_hbm.at[c, pl.ds(NS * RPT, REM)])


_sc_agg = pl.kernel(
    _sc_agg_body,
    out_type=jax.ShapeDtypeStruct((NC, N, DH), jnp.float32),
    mesh=plsc.VectorSubcoreMesh(core_axis_name="c", subcore_axis_name="s"),
    scratch_types=[
        pltpu.VMEM((EPT,), jnp.int32),
        pltpu.VMEM((CHUNK,), jnp.int32),
        pltpu.VMEM((CHUNK,), jnp.int32),
        pltpu.VMEM((2, CHUNK, DH), jnp.float32),
        pltpu.VMEM_SHARED((N, DH), jnp.float32),
        pltpu.SemaphoreType.DMA,
        pltpu.SemaphoreType.DMA,
        pltpu.SemaphoreType.DMA,
        pltpu.SemaphoreType.DMA,
    ],
)


BM = 1000


def _mm_body(h_ref, a0_ref, a1_ref, w_ref, b_ref, o_ref):
    agg = jnp.concatenate([a0_ref[...], a1_ref[...]], axis=1)
    s = h_ref[...] + agg
    o_ref[...] = (
        jnp.dot(s, w_ref[...], preferred_element_type=jnp.float32) + b_ref[...]
    )


@functools.partial(jax.jit, static_argnames=())
def _tc_mm(h, a0, a1, w, b2d):
    return pl.pallas_call(
        _mm_body,
        grid=(N // BM,),
        in_specs=[
            pl.BlockSpec((BM, D), lambda i: (i, 0)),
            pl.BlockSpec((BM, DH), lambda i: (i, 0)),
            pl.BlockSpec((BM, DH), lambda i: (i, 0)),
            pl.BlockSpec((D, D), lambda i: (0, 0)),
            pl.BlockSpec((1, D), lambda i: (0, 0)),
        ],
        out_specs=pl.BlockSpec((BM, D), lambda i: (i, 0)),
        out_shape=jax.ShapeDtypeStruct((N, D), jnp.float32),
    )(h, a0, a1, w, b2d)


def kernel(x, edge_index, W0, b0, W1, b1, W2, b2):
    src = edge_index[0].astype(jnp.int32)
    dst = edge_index[1].astype(jnp.int32)
    src2 = jnp.concatenate([src * 2, src * 2 + 1])
    zeros = jnp.zeros((RPT, DH), jnp.float32)

    h = x
    for W, b in ((W0, b0), (W1, b1), (W2, b2)):
        h2 = h.reshape(2 * N, DH)
        agg = _sc_agg(src2, dst, h2, zeros)
        h = _tc_mm(h, agg[0], agg[1], W, b.reshape(1, D))
    return h


# speedup vs baseline: 9.1608x; 1.2516x over previous
import functools

import jax
import jax.numpy as jnp
from jax import lax
from jax.experimental import pallas as pl
from jax.experimental.pallas import tpu as pltpu
from jax.experimental.pallas import tpu_sc as plsc

N = 10000
E = 160000
D = 256
DH = 128

_INFO = plsc.get_sparse_core_info()
NC = _INFO.num_cores
NS = _INFO.num_subcores
EPT = E // NS
RPT = 624
REM = N - NS * RPT
CHUNK = 64
NCH = EPT // CHUNK
TAIL = EPT - NCH * CHUNK


NBUF = 4


def _sc_agg_body(src2_hbm, dst_hbm, h2_hbm, out_hbm,
                 idx_s, db0, db1, db2, db3, dbt, rowst, rows, acc,
                 sg0, sg1, sg2, sg3, sd0, sd1, sd2, sd3,
                 ss0, ss1, ss2, ss3, semt):
    c = lax.axis_index("c")
    s = lax.axis_index("s")
    sg = (sg0, sg1, sg2, sg3)
    sd = (sd0, sd1, sd2, sd3)
    ss = (ss0, ss1, ss2, ss3)
    db = (db0, db1, db2, db3)

    zv = jnp.zeros((16,), jnp.float32)

    def fill(r, carry):
        for j in range(DH // 16):
            rows[0, r, pl.ds(16 * j, 16)] = zv
        return carry

    lax.fori_loop(0, CHUNK, fill, 0)
    for q in range(RPT // CHUNK):
        pltpu.sync_copy(rows.at[0, pl.ds(0, CHUNK)],
                        acc.at[pl.ds(s * RPT + q * CHUNK, CHUNK)])
    _REM0 = RPT % CHUNK
    if _REM0:
        pltpu.sync_copy(rows.at[0, pl.ds(0, _REM0)],
                        acc.at[pl.ds(s * RPT + RPT - _REM0, _REM0)])

    @pl.when(s == NS - 1)
    def _():
        pltpu.sync_copy(rows.at[0, pl.ds(0, REM)],
                        acc.at[pl.ds(NS * RPT, REM)])

    pltpu.sync_copy(src2_hbm.at[pl.ds(c * E + s * EPT, EPT)], idx_s)
    plsc.subcore_barrier()

    def gather_start(i, b):
        pltpu.async_copy(h2_hbm.at[idx_s.at[pl.ds(i * CHUNK, CHUNK)]],
                         rows.at[b], sg[b])

    def gather_wait(b):
        pltpu.make_async_copy(h2_hbm.at[idx_s.at[pl.ds(0, CHUNK)]],
                              rows.at[b], sg[b]).wait()

    def dst_start(i, b):
        pltpu.async_copy(dst_hbm.at[pl.ds(s * EPT + i * CHUNK, CHUNK)],
                         db[b], sd[b])

    def dst_wait(b):
        pltpu.make_async_copy(dst_hbm.at[pl.ds(0, CHUNK)], db[b],
                              sd[b]).wait()

    def scatter_start(b):
        pltpu.async_copy(rows.at[b], acc.at[db[b]], ss[b], add=True)

    def scatter_wait(b):
        pltpu.make_async_copy(rows.at[b], acc.at[db[b]], ss[b]).wait()

    for i in range(NBUF - 1):
        dst_start(i, i)
        gather_start(i, i)

    def slot(i, b, wait_prev=True):
        gather_wait(b)
        dst_wait(b)
        scatter_start(b)
        if wait_prev:
            scatter_wait((b + NBUF - 1) % NBUF)

        @pl.when(i + NBUF - 1 < NCH)
        def _():
            dst_start(i + NBUF - 1, (b + NBUF - 1) % NBUF)
            gather_start(i + NBUF - 1, (b + NBUF - 1) % NBUF)

    for i in range(NBUF):
        slot(i, i % NBUF, wait_prev=(i > 0))

    def step(k, carry):
        i0 = NBUF * k + NBUF
        for j in range(NBUF):
            slot(i0 + j, j)
        return carry

    lax.fori_loop(0, (NCH - NBUF) // NBUF, step, 0)
    scatter_wait((NCH - 1) % NBUF)

    tb = NCH * CHUNK
    pltpu.sync_copy(dst_hbm.at[pl.ds(s * EPT + tb, TAIL)], dbt)
    pltpu.async_copy(h2_hbm.at[idx_s.at[pl.ds(tb, TAIL)]], rowst, semt).wait()
    pltpu.sync_copy(rowst, acc.at[dbt], add=True)

    plsc.subcore_barrier()
    pltpu.sync_copy(acc.at[pl.ds(s * RPT, RPT)],
                    out_hbm.at[c, pl.ds(s * RPT, RPT)])

    @pl.when(s == NS - 1)
    def _():
        pltpu.sync_copy(acc.at[pl.ds(NS * RPT, REM)],
                        out_hbm.at[c, pl.ds(NS * RPT, REM)])


_sc_agg = pl.kernel(
    _sc_agg_body,
    out_type=jax.ShapeDtypeStruct((NC, N, DH), jnp.float32),
    mesh=plsc.VectorSubcoreMesh(core_axis_name="c", subcore_axis_name="s"),
    scratch_types=(
        [pltpu.VMEM((EPT,), jnp.int32)]
        + [pltpu.VMEM((CHUNK,), jnp.int32) for _ in range(NBUF)]
        + [pltpu.VMEM((TAIL,), jnp.int32),
           pltpu.VMEM((TAIL, DH), jnp.float32),
           pltpu.VMEM((NBUF, CHUNK, DH), jnp.float32),
           pltpu.VMEM_SHARED((N, DH), jnp.float32)]
        + [pltpu.SemaphoreType.DMA for _ in range(3 * NBUF + 1)]
    ),
)


BM = 1000


def _sum_cat(h_ref, a_ref):
    return (jnp.concatenate([h_ref[0], h_ref[1]], axis=1)
            + jnp.concatenate([a_ref[0], a_ref[1]], axis=1))


def _mm_mid_body(h_ref, a_ref, w_ref, b_ref, o_ref):
    r = jnp.dot(_sum_cat(h_ref, a_ref), w_ref[...],
                preferred_element_type=jnp.float32) + b_ref[...]
    o_ref[0] = r[:, :DH]
    o_ref[1] = r[:, DH:]


def _mm_last_body(h_ref, a_ref, w_ref, b_ref, o_ref):
    o_ref[...] = jnp.dot(_sum_cat(h_ref, a_ref), w_ref[...],
                         preferred_element_type=jnp.float32) + b_ref[...]


_PLANE_SPECS = [
    pl.BlockSpec((NC, BM, DH), lambda i: (0, i, 0)),
    pl.BlockSpec((NC, BM, DH), lambda i: (0, i, 0)),
    pl.BlockSpec((D, D), lambda i: (0, 0)),
    pl.BlockSpec((1, D), lambda i: (0, 0)),
]


def _tc_mm_mid(h2, agg, w, b2d):
    return pl.pallas_call(
        _mm_mid_body,
        grid=(N // BM,),
        in_specs=_PLANE_SPECS,
        out_specs=pl.BlockSpec((NC, BM, DH), lambda i: (0, i, 0)),
        out_shape=jax.ShapeDtypeStruct((NC, N, DH), jnp.float32),
    )(h2, agg, w, b2d)


def _tc_mm_last(h2, agg, w, b2d):
    return pl.pallas_call(
        _mm_last_body,
        grid=(N // BM,),
        in_specs=_PLANE_SPECS,
        out_specs=pl.BlockSpec((BM, D), lambda i: (i, 0)),
        out_shape=jax.ShapeDtypeStruct((N, D), jnp.float32),
    )(h2, agg, w, b2d)


def kernel(x, edge_index, W0, b0, W1, b1, W2, b2):
    src = edge_index[0].astype(jnp.int32)
    dst = edge_index[1].astype(jnp.int32)
    src2 = jnp.concatenate([src, src + N])

    h2 = jnp.stack([x[:, :DH], x[:, DH:]])
    for W, b in ((W0, b0), (W1, b1)):
        agg = _sc_agg(src2, dst, h2.reshape(NC * N, DH))
        h2 = _tc_mm_mid(h2, agg, W, b.reshape(1, D))
    agg = _sc_agg(src2, dst, h2.reshape(NC * N, DH))
    return _tc_mm_last(h2, agg, W2, b2.reshape(1, D))


# speedup vs baseline: 9.5106x; 1.0382x over previous
import jax
import jax.numpy as jnp
from jax import lax
from jax.experimental import pallas as pl
from jax.experimental.pallas import tpu as pltpu
from jax.experimental.pallas import tpu_sc as plsc

N = 10000
E = 160000
D = 256
DH = 128

_INFO = plsc.get_sparse_core_info()
NC = _INFO.num_cores
NS = _INFO.num_subcores
EPT = E // NS
RPT = 624
REM = N - NS * RPT
CHUNK = 96
NCH = EPT // CHUNK
TAIL = EPT - NCH * CHUNK
NBUF = 3

BF = jnp.float32


def _sc_agg_body(src2_hbm, dst_hbm, hb_hbm, out_hbm,
                 idx_s, db0, db1, db2, dbt, rowst, rows, acc,
                 sg0, sg1, sg2, sd0, sd1, sd2, ss0, ss1, ss2, semt):
    c = lax.axis_index("c")
    s = lax.axis_index("s")
    sg = (sg0, sg1, sg2)
    sd = (sd0, sd1, sd2)
    ss = (ss0, ss1, ss2)
    db = (db0, db1, db2)

    zv = jnp.zeros((16,), BF)

    def fill(r, carry):
        for j in range(DH // 16):
            rows[0, r, pl.ds(16 * j, 16)] = zv
        return carry

    lax.fori_loop(0, CHUNK, fill, 0)
    for q in range(RPT // CHUNK):
        pltpu.sync_copy(rows.at[0, pl.ds(0, CHUNK)],
                        acc.at[pl.ds(s * RPT + q * CHUNK, CHUNK)])
    _R0 = RPT % CHUNK
    if _R0:
        pltpu.sync_copy(rows.at[0, pl.ds(0, _R0)],
                        acc.at[pl.ds(s * RPT + RPT - _R0, _R0)])

    @pl.when(s == NS - 1)
    def _():
        pltpu.sync_copy(rows.at[0, pl.ds(0, REM)],
                        acc.at[pl.ds(NS * RPT, REM)])

    pltpu.sync_copy(src2_hbm.at[pl.ds(c * E + s * EPT, EPT)], idx_s)
    plsc.subcore_barrier()

    def gather_start(i, b):
        pltpu.async_copy(hb_hbm.at[idx_s.at[pl.ds(i * CHUNK, CHUNK)]],
                         rows.at[b], sg[b])

    def gather_wait(b):
        pltpu.make_async_copy(hb_hbm.at[idx_s.at[pl.ds(0, CHUNK)]],
                              rows.at[b], sg[b]).wait()

    def dst_start(i, b):
        pltpu.async_copy(dst_hbm.at[pl.ds(s * EPT + i * CHUNK, CHUNK)],
                         db[b], sd[b])

    def dst_wait(b):
        pltpu.make_async_copy(dst_hbm.at[pl.ds(0, CHUNK)], db[b],
                              sd[b]).wait()

    def scatter_start(b):
        pltpu.async_copy(rows.at[b], acc.at[db[b]], ss[b], add=True)

    def scatter_wait(b):
        pltpu.make_async_copy(rows.at[b], acc.at[db[b]], ss[b]).wait()

    for i in range(NBUF - 1):
        dst_start(i, i)
        gather_start(i, i)

    def slot(i, b, wait_prev=True):
        gather_wait(b)
        dst_wait(b)
        scatter_start(b)
        if wait_prev:
            scatter_wait((b + NBUF - 1) % NBUF)

        @pl.when(i + NBUF - 1 < NCH)
        def _():
            dst_start(i + NBUF - 1, (b + NBUF - 1) % NBUF)
            gather_start(i + NBUF - 1, (b + NBUF - 1) % NBUF)

    for i in range(NBUF):
        slot(i, i % NBUF, wait_prev=(i > 0))

    def step(k, carry):
        i0 = NBUF * k + NBUF
        for j in range(NBUF):
            slot(i0 + j, j)
        return carry

    main = (NCH - NBUF) // NBUF
    lax.fori_loop(0, main, step, 0)
    for i in range(NBUF * (main + 1), NCH):
        slot(i, i % NBUF)
    scatter_wait((NCH - 1) % NBUF)

    tb = NCH * CHUNK
    pltpu.sync_copy(dst_hbm.at[pl.ds(s * EPT + tb, TAIL)], dbt)
    pltpu.async_copy(hb_hbm.at[idx_s.at[pl.ds(tb, TAIL)]], rowst, semt).wait()
    pltpu.sync_copy(rowst, acc.at[dbt], add=True)

    plsc.subcore_barrier()
    pltpu.sync_copy(acc.at[pl.ds(s * RPT, RPT)],
                    out_hbm.at[c, pl.ds(s * RPT, RPT)])

    @pl.when(s == NS - 1)
    def _():
        pltpu.sync_copy(acc.at[pl.ds(NS * RPT, REM)],
                        out_hbm.at[c, pl.ds(NS * RPT, REM)])


_sc_agg = pl.kernel(
    _sc_agg_body,
    out_type=jax.ShapeDtypeStruct((NC, N, DH), BF),
    mesh=plsc.VectorSubcoreMesh(core_axis_name="c", subcore_axis_name="s"),
    scratch_types=(
        [pltpu.VMEM((EPT,), jnp.int32)]
        + [pltpu.VMEM((CHUNK,), jnp.int32) for _ in range(NBUF)]
        + [pltpu.VMEM((TAIL,), jnp.int32),
           pltpu.VMEM((TAIL, DH), BF),
           pltpu.VMEM((NBUF, CHUNK, DH), BF),
           pltpu.VMEM_SHARED((N, DH), BF)]
        + [pltpu.SemaphoreType.DMA for _ in range(3 * NBUF + 1)]
    ),
)


BM = 1000


def _sum_cat(h_ref, a_ref):
    h = jnp.concatenate([h_ref[0], h_ref[1]], axis=1)
    a = jnp.concatenate([a_ref[0], a_ref[1]], axis=1)
    return h + a


def _mm_mid_body(h_ref, a_ref, w_ref, b_ref, o_ref):
    r = jnp.dot(_sum_cat(h_ref, a_ref), w_ref[...],
                preferred_element_type=jnp.float32) + b_ref[...]
    o_ref[0] = r[:, :DH]
    o_ref[1] = r[:, DH:]


def _mm_last_body(h_ref, a_ref, w_ref, b_ref, o_ref):
    o_ref[...] = jnp.dot(_sum_cat(h_ref, a_ref), w_ref[...],
                         preferred_element_type=jnp.float32) + b_ref[...]


_PLANE_SPECS = [
    pl.BlockSpec((NC, BM, DH), lambda i: (0, i, 0)),
    pl.BlockSpec((NC, BM, DH), lambda i: (0, i, 0)),
    pl.BlockSpec((D, D), lambda i: (0, 0)),
    pl.BlockSpec((1, D), lambda i: (0, 0)),
]


def _tc_mm_mid(h2, agg, w, b2d):
    return pl.pallas_call(
        _mm_mid_body,
        grid=(N // BM,),
        in_specs=_PLANE_SPECS,
        out_specs=pl.BlockSpec((NC, BM, DH), lambda i: (0, i, 0)),
        out_shape=jax.ShapeDtypeStruct((NC, N, DH), jnp.float32),
    )(h2, agg, w, b2d)


def _tc_mm_last(h2, agg, w, b2d):
    return pl.pallas_call(
        _mm_last_body,
        grid=(N // BM,),
        in_specs=_PLANE_SPECS,
        out_specs=pl.BlockSpec((BM, D), lambda i: (i, 0)),
        out_shape=jax.ShapeDtypeStruct((N, D), jnp.float32),
    )(h2, agg, w, b2d)


def kernel(x, edge_index, W0, b0, W1, b1, W2, b2):
    src = edge_index[0].astype(jnp.int32)
    dst = edge_index[1].astype(jnp.int32)
    src2 = jnp.concatenate([src, src + N])

    h2 = jnp.stack([x[:, :DH], x[:, DH:]])
    for W, b in ((W0, b0), (W1, b1)):
        agg = _sc_agg(src2, dst, h2.reshape(NC * N, DH))
        h2 = _tc_mm_mid(h2, agg, W, b.reshape(1, D))
    agg = _sc_agg(src2, dst, h2.reshape(NC * N, DH))
    return _tc_mm_last(h2, agg, W2, b2.reshape(1, D))


# speedup vs baseline: 9.5311x; 1.0022x over previous
import jax
import jax.numpy as jnp
from jax import lax
from jax.experimental import pallas as pl
from jax.experimental.pallas import tpu as pltpu
from jax.experimental.pallas import tpu_sc as plsc

N = 10000
E = 160000
D = 256
DH = 128

_INFO = plsc.get_sparse_core_info()
NC = _INFO.num_cores
NS = _INFO.num_subcores
EPT = E // NS
RPT = 624
REM = N - NS * RPT
CHUNK = 96
NCH = EPT // CHUNK
TAIL = EPT - NCH * CHUNK
NBUF = 3

BF = jnp.float32


def _sc_agg_body(src2_hbm, dst_hbm, hb_hbm, out_hbm,
                 idx_s, db0, db1, db2, dbt, rowst, rows, acc,
                 sg0, sg1, sg2, sd0, sd1, sd2, ss0, ss1, ss2, semt):
    c = lax.axis_index("c")
    s = lax.axis_index("s")
    sg = (sg0, sg1, sg2)
    sd = (sd0, sd1, sd2)
    ss = (ss0, ss1, ss2)
    db = (db0, db1, db2)

    zv = jnp.zeros((16,), BF)

    def fill(r, carry):
        for j in range(DH // 16):
            rows[0, r, pl.ds(16 * j, 16)] = zv
        return carry

    lax.fori_loop(0, CHUNK, fill, 0)
    for q in range(RPT // CHUNK):
        pltpu.sync_copy(rows.at[0, pl.ds(0, CHUNK)],
                        acc.at[pl.ds(s * RPT + q * CHUNK, CHUNK)])
    _R0 = RPT % CHUNK
    if _R0:
        pltpu.sync_copy(rows.at[0, pl.ds(0, _R0)],
                        acc.at[pl.ds(s * RPT + RPT - _R0, _R0)])

    @pl.when(s == NS - 1)
    def _():
        pltpu.sync_copy(rows.at[0, pl.ds(0, REM)],
                        acc.at[pl.ds(NS * RPT, REM)])

    pltpu.sync_copy(src2_hbm.at[pl.ds(c * E + s * EPT, EPT)], idx_s)
    plsc.subcore_barrier()

    def gather_start(i, b):
        pltpu.async_copy(hb_hbm.at[idx_s.at[pl.ds(i * CHUNK, CHUNK)]],
                         rows.at[b], sg[b])

    def gather_wait(b):
        pltpu.make_async_copy(hb_hbm.at[idx_s.at[pl.ds(0, CHUNK)]],
                              rows.at[b], sg[b]).wait()

    def dst_start(i, b):
        pltpu.async_copy(dst_hbm.at[pl.ds(s * EPT + i * CHUNK, CHUNK)],
                         db[b], sd[b])

    def dst_wait(b):
        pltpu.make_async_copy(dst_hbm.at[pl.ds(0, CHUNK)], db[b],
                              sd[b]).wait()

    def scatter_start(b):
        pltpu.async_copy(rows.at[b], acc.at[db[b]], ss[b], add=True, priority=1)

    def scatter_wait(b):
        pltpu.make_async_copy(rows.at[b], acc.at[db[b]], ss[b]).wait()

    for i in range(NBUF - 1):
        dst_start(i, i)
        gather_start(i, i)

    def slot(i, b, wait_prev=True):
        gather_wait(b)
        dst_wait(b)
        scatter_start(b)
        if wait_prev:
            scatter_wait((b + NBUF - 1) % NBUF)

        @pl.when(i + NBUF - 1 < NCH)
        def _():
            dst_start(i + NBUF - 1, (b + NBUF - 1) % NBUF)
            gather_start(i + NBUF - 1, (b + NBUF - 1) % NBUF)

    for i in range(NBUF):
        slot(i, i % NBUF, wait_prev=(i > 0))

    def step(k, carry):
        i0 = NBUF * k + NBUF
        for j in range(NBUF):
            slot(i0 + j, j)
        return carry

    main = (NCH - NBUF) // NBUF
    lax.fori_loop(0, main, step, 0)
    for i in range(NBUF * (main + 1), NCH):
        slot(i, i % NBUF)
    scatter_wait((NCH - 1) % NBUF)

    tb = NCH * CHUNK
    pltpu.sync_copy(dst_hbm.at[pl.ds(s * EPT + tb, TAIL)], dbt)
    pltpu.async_copy(hb_hbm.at[idx_s.at[pl.ds(tb, TAIL)]], rowst, semt).wait()
    pltpu.sync_copy(rowst, acc.at[dbt], add=True)

    plsc.subcore_barrier()
    pltpu.sync_copy(acc.at[pl.ds(s * RPT, RPT)],
                    out_hbm.at[c, pl.ds(s * RPT, RPT)])

    @pl.when(s == NS - 1)
    def _():
        pltpu.sync_copy(acc.at[pl.ds(NS * RPT, REM)],
                        out_hbm.at[c, pl.ds(NS * RPT, REM)])


_sc_agg = pl.kernel(
    _sc_agg_body,
    out_type=jax.ShapeDtypeStruct((NC, N, DH), BF),
    mesh=plsc.VectorSubcoreMesh(core_axis_name="c", subcore_axis_name="s"),
    scratch_types=(
        [pltpu.VMEM((EPT,), jnp.int32)]
        + [pltpu.VMEM((CHUNK,), jnp.int32) for _ in range(NBUF)]
        + [pltpu.VMEM((TAIL,), jnp.int32),
           pltpu.VMEM((TAIL, DH), BF),
           pltpu.VMEM((NBUF, CHUNK, DH), BF),
           pltpu.VMEM_SHARED((N, DH), BF)]
        + [pltpu.SemaphoreType.DMA for _ in range(3 * NBUF + 1)]
    ),
)


BM = 1000


def _sum_cat(h_ref, a_ref):
    h = jnp.concatenate([h_ref[0], h_ref[1]], axis=1)
    a = jnp.concatenate([a_ref[0], a_ref[1]], axis=1)
    return h + a


def _mm_mid_body(h_ref, a_ref, w_ref, b_ref, o_ref):
    r = jnp.dot(_sum_cat(h_ref, a_ref), w_ref[...],
                preferred_element_type=jnp.float32) + b_ref[...]
    o_ref[0] = r[:, :DH]
    o_ref[1] = r[:, DH:]


def _mm_last_body(h_ref, a_ref, w_ref, b_ref, o_ref):
    o_ref[...] = jnp.dot(_sum_cat(h_ref, a_ref), w_ref[...],
                         preferred_element_type=jnp.float32) + b_ref[...]


_PLANE_SPECS = [
    pl.BlockSpec((NC, BM, DH), lambda i: (0, i, 0)),
    pl.BlockSpec((NC, BM, DH), lambda i: (0, i, 0)),
    pl.BlockSpec((D, D), lambda i: (0, 0)),
    pl.BlockSpec((1, D), lambda i: (0, 0)),
]


def _tc_mm_mid(h2, agg, w, b2d):
    return pl.pallas_call(
        _mm_mid_body,
        grid=(N // BM,),
        in_specs=_PLANE_SPECS,
        out_specs=pl.BlockSpec((NC, BM, DH), lambda i: (0, i, 0)),
        out_shape=jax.ShapeDtypeStruct((NC, N, DH), jnp.float32),
    )(h2, agg, w, b2d)


def _tc_mm_last(h2, agg, w, b2d):
    return pl.pallas_call(
        _mm_last_body,
        grid=(N // BM,),
        in_specs=_PLANE_SPECS,
        out_specs=pl.BlockSpec((BM, D), lambda i: (i, 0)),
        out_shape=jax.ShapeDtypeStruct((N, D), jnp.float32),
    )(h2, agg, w, b2d)


def kernel(x, edge_index, W0, b0, W1, b1, W2, b2):
    src = edge_index[0].astype(jnp.int32)
    dst = edge_index[1].astype(jnp.int32)
    src2 = jnp.concatenate([src, src + N])

    h2 = jnp.stack([x[:, :DH], x[:, DH:]])
    for W, b in ((W0, b0), (W1, b1)):
        agg = _sc_agg(src2, dst, h2.reshape(NC * N, DH))
        h2 = _tc_mm_mid(h2, agg, W, b.reshape(1, D))
    agg = _sc_agg(src2, dst, h2.reshape(NC * N, DH))
    return _tc_mm_last(h2, agg, W2, b2.reshape(1, D))
